# baseline (speedup 1.0000x reference)
import functools

import jax
import jax.numpy as jnp
from jax import lax
from jax.experimental import pallas as pl
from jax.experimental.pallas import tpu as pltpu
from jax.experimental.pallas import tpu_sc as plsc

_B = 16384
_D = 512
_DW = _D // 2
_NW = 32
_ROWS = 4 * _B
_BPW = _ROWS // _NW
_CHUNK = 64
_NCHUNK = _BPW // _CHUNK
_VROWS = 74

_mesh = plsc.VectorSubcoreMesh(core_axis_name="c", subcore_axis_name="s")


@functools.partial(
    pl.kernel,
    mesh=_mesh,
    out_type=jax.ShapeDtypeStruct((_ROWS, _D), jnp.float32),
    scratch_types=[
        pltpu.VMEM((_BPW,), jnp.int32),
        pltpu.VMEM((_CHUNK, _DW), jnp.int32),
        pltpu.VMEM((_CHUNK, _DW), jnp.int32),
        pltpu.VMEM((_CHUNK, _DW), jnp.int32),
        pltpu.VMEM((_CHUNK, _D), jnp.float32),
        pltpu.VMEM((_CHUNK, _D), jnp.float32),
        pltpu.SemaphoreType.DMA,
        pltpu.SemaphoreType.DMA,
    ],
    compiler_params=pltpu.CompilerParams(needs_layout_passes=False),
)
def _gather_all(table_hbm, idx_hbm, out_hbm, idx_v, p0, p1, p2, f0, f1, gsem, ssem):
    pbufs = (p0, p1, p2)
    fbufs = (f0, f1)
    wid = lax.axis_index("s") * 2 + lax.axis_index("c")
    base = wid * _BPW

    pltpu.sync_copy(idx_hbm.at[pl.ds(base, _BPW)], idx_v)

    def fire_gather(g, b):
        pltpu.async_copy(
            table_hbm.at[idx_v.at[pl.ds(g * _CHUNK, _CHUNK)]], pbufs[b], gsem
        )

    def wait_gather(b):
        pltpu.make_async_copy(table_hbm.at[pl.ds(0, _CHUNK)], pbufs[b], gsem).wait()

    def fire_scatter(g, b):
        pltpu.async_copy(fbufs[b], out_hbm.at[pl.ds(base + g * _CHUNK, _CHUNK)], ssem)

    def wait_scatter(b):
        pltpu.make_async_copy(fbufs[b], out_hbm.at[pl.ds(base, _CHUNK)], ssem).wait()

    def convert(vb, fb):
        hi_mask = jnp.int32(-65536)

        def col_body(k, carry):
            col = k * 16
            for r in range(_CHUNK):
                w = pbufs[vb][r, pl.ds(col, 16)]
                lo = plsc.bitcast(w << 16, jnp.float32)
                hi = plsc.bitcast(w & hi_mask, jnp.float32)
                fbufs[fb][r, pl.ds(col, 16)] = lo
                fbufs[fb][r, pl.ds(_DW + col, 16)] = hi
            return carry

        lax.fori_loop(0, _DW // 16, col_body, 0)

    def slot(g, vb, fb, drain, refill):
        wait_gather(vb)
        if drain:
            wait_scatter(fb)
        convert(vb, fb)
        fire_scatter(g, fb)
        if refill:
            fire_gather(g + 3, vb)

    fire_gather(0, 0)
    fire_gather(1, 1)
    fire_gather(2, 2)
    for g in range(6):
        slot(g, g % 3, g % 2, g >= 2, True)

    def body(r, carry):
        for j in range(6):
            slot(6 + r * 6 + j, j % 3, j % 2, True, True)
        return carry

    lax.fori_loop(0, (_NCHUNK - 14) // 6, body, 0)

    for g in range(_NCHUNK - 8, _NCHUNK):
        slot(g, g % 3, g % 2, True, g + 3 < _NCHUNK)
    wait_scatter(0)
    wait_scatter(1)


def kernel(hour, day, weekday, month, W_hour, W_day, W_weekday, W_month):
    table = jnp.concatenate([W_hour, W_day, W_weekday, W_month], axis=0)
    tb = table.astype(jnp.bfloat16)
    packed = jax.lax.bitcast_convert_type(
        jnp.stack([tb[:, :_DW], tb[:, _DW:]], axis=-1), jnp.int32
    )
    packed_rep = jnp.tile(packed, (_NW, 1))
    idx = jnp.stack(
        [
            hour.astype(jnp.int32),
            day.astype(jnp.int32) + 24,
            weekday.astype(jnp.int32) + 55,
            month.astype(jnp.int32) + 62,
        ],
        axis=1,
    ).reshape(_ROWS)
    idx = idx + (jnp.arange(_ROWS, dtype=jnp.int32) // _BPW) * _VROWS
    out = _gather_all(packed_rep, idx)
    return out.reshape(_B, 4 * _D)


# speedup vs baseline: 1.0073x; 1.0073x over previous
import functools

import jax
import jax.numpy as jnp
from jax import lax
from jax.experimental import pallas as pl
from jax.experimental.pallas import tpu as pltpu
from jax.experimental.pallas import tpu_sc as plsc

_B = 16384
_D = 512
_DW = _D // 2
_NW = 32
_ROWS = 4 * _B
_BPW = _ROWS // _NW
_CHUNK = 32
_NCHUNK = _BPW // _CHUNK
_VROWS = 74

_mesh = plsc.VectorSubcoreMesh(core_axis_name="c", subcore_axis_name="s")


@functools.partial(
    pl.kernel,
    mesh=_mesh,
    out_type=jax.ShapeDtypeStruct((_ROWS, _D), jnp.float32),
    scratch_types=[
        pltpu.VMEM((_BPW,), jnp.int32),
        pltpu.VMEM((_CHUNK, _DW), jnp.int32),
        pltpu.VMEM((_CHUNK, _DW), jnp.int32),
        pltpu.VMEM((_CHUNK, _D), jnp.float32),
        pltpu.VMEM((_CHUNK, _D), jnp.float32),
        pltpu.SemaphoreType.DMA,
        pltpu.SemaphoreType.DMA,
    ],
    compiler_params=pltpu.CompilerParams(needs_layout_passes=False),
)
def _gather_all(table_hbm, idx_hbm, out_hbm, idx_v, p0, p1, f0, f1, gsem, ssem):
    pbufs = (p0, p1)
    fbufs = (f0, f1)
    wid = lax.axis_index("s") * 2 + lax.axis_index("c")
    base = wid * _BPW

    pltpu.sync_copy(idx_hbm.at[pl.ds(base, _BPW)], idx_v)

    def fire_gather(g, b):
        pltpu.async_copy(
            table_hbm.at[idx_v.at[pl.ds(g * _CHUNK, _CHUNK)]], pbufs[b], gsem
        )

    def wait_gather(b):
        pltpu.make_async_copy(table_hbm.at[pl.ds(0, _CHUNK)], pbufs[b], gsem).wait()

    def fire_scatter(g, b):
        pltpu.async_copy(fbufs[b], out_hbm.at[pl.ds(base + g * _CHUNK, _CHUNK)], ssem)

    def wait_scatter(b):
        pltpu.make_async_copy(fbufs[b], out_hbm.at[pl.ds(base, _CHUNK)], ssem).wait()

    def convert(b):
        hi_mask = jnp.int32(-65536)

        def col_body(k, carry):
            col = k * 16
            for r in range(_CHUNK):
                w = pbufs[b][r, pl.ds(col, 16)]
                lo = plsc.bitcast(w << 16, jnp.float32)
                hi = plsc.bitcast(w & hi_mask, jnp.float32)
                fbufs[b][r, pl.ds(col, 16)] = lo
                fbufs[b][r, pl.ds(_DW + col, 16)] = hi
            return carry

        lax.fori_loop(0, _DW // 16, col_body, 0)

    def slot(g, b, drain, refill):
        wait_gather(b)
        if drain:
            wait_scatter(b)
        convert(b)
        fire_scatter(g, b)
        if refill:
            fire_gather(g + 2, b)

    fire_gather(0, 0)
    fire_gather(1, 1)
    slot(0, 0, False, True)
    slot(1, 1, False, True)

    def body(r, carry):
        for j in range(2):
            slot(2 + r * 2 + j, j, True, True)
        return carry

    lax.fori_loop(0, (_NCHUNK - 4) // 2, body, 0)

    slot(_NCHUNK - 2, 0, True, False)
    slot(_NCHUNK - 1, 1, True, False)
    wait_scatter(0)
    wait_scatter(1)


def kernel(hour, day, weekday, month, W_hour, W_day, W_weekday, W_month):
    table = jnp.concatenate([W_hour, W_day, W_weekday, W_month], axis=0)
    tb = table.astype(jnp.bfloat16)
    packed = jax.lax.bitcast_convert_type(
        jnp.stack([tb[:, :_DW], tb[:, _DW:]], axis=-1), jnp.int32
    )
    packed_rep = jnp.tile(packed, (_NW, 1))
    idx = jnp.stack(
        [
            hour.astype(jnp.int32),
            day.astype(jnp.int32) + 24,
            weekday.astype(jnp.int32) + 55,
            month.astype(jnp.int32) + 62,
        ],
        axis=1,
    ).reshape(_ROWS)
    idx = idx + (jnp.arange(_ROWS, dtype=jnp.int32) // _BPW) * _VROWS
    out = _gather_all(packed_rep, idx)
    return out.reshape(_B, 4 * _D)


# speedup vs baseline: 1.0127x; 1.0054x over previous
import functools

import jax
import jax.numpy as jnp
from jax import lax
from jax.experimental import pallas as pl
from jax.experimental.pallas import tpu as pltpu
from jax.experimental.pallas import tpu_sc as plsc

_B = 16384
_D = 512
_DW = _D // 2
_NW = 32
_ROWS = 4 * _B
_BPW = _ROWS // _NW
_CHUNK = 64
_NCHUNK = _BPW // _CHUNK
_VROWS = 74

_mesh = plsc.VectorSubcoreMesh(core_axis_name="c", subcore_axis_name="s")


@functools.partial(
    pl.kernel,
    mesh=_mesh,
    out_type=jax.ShapeDtypeStruct((_ROWS, _D), jnp.float32),
    scratch_types=[
        pltpu.VMEM((_BPW,), jnp.int32),
        pltpu.VMEM((_CHUNK, _DW), jnp.int32),
        pltpu.VMEM((_CHUNK, _DW), jnp.int32),
        pltpu.VMEM((_CHUNK, _D), jnp.float32),
        pltpu.VMEM((_CHUNK, _D), jnp.float32),
        pltpu.SemaphoreType.DMA,
        pltpu.SemaphoreType.DMA,
    ],
    compiler_params=pltpu.CompilerParams(needs_layout_passes=False),
)
def _gather_all(table_hbm, idx_hbm, out_hbm, idx_v, p0, p1, f0, f1, gsem, ssem):
    pbufs = (p0, p1)
    fbufs = (f0, f1)
    wid = lax.axis_index("s") * 2 + lax.axis_index("c")
    base = wid * _BPW

    pltpu.sync_copy(idx_hbm.at[pl.ds(base, _BPW)], idx_v)

    def fire_gather(g, b):
        pltpu.async_copy(
            table_hbm.at[idx_v.at[pl.ds(g * _CHUNK, _CHUNK)]], pbufs[b], gsem
        )

    def wait_gather(b):
        pltpu.make_async_copy(table_hbm.at[pl.ds(0, _CHUNK)], pbufs[b], gsem).wait()

    def fire_scatter(g, b):
        pltpu.async_copy(fbufs[b], out_hbm.at[pl.ds(base + g * _CHUNK, _CHUNK)], ssem)

    def wait_scatter(b):
        pltpu.make_async_copy(fbufs[b], out_hbm.at[pl.ds(base, _CHUNK)], ssem).wait()

    def convert(b):
        hi_mask = jnp.int32(-65536)

        def col_body(k, carry):
            col = k * 16
            for r in range(_CHUNK):
                w = pbufs[b][r, pl.ds(col, 16)]
                lo = plsc.bitcast(w << 16, jnp.float32)
                hi = plsc.bitcast(w & hi_mask, jnp.float32)
                fbufs[b][r, pl.ds(col, 16)] = lo
                fbufs[b][r, pl.ds(_DW + col, 16)] = hi
            return carry

        lax.fori_loop(0, _DW // 16, col_body, 0)

    def slot(g, b, drain, refill):
        wait_gather(b)
        if drain:
            wait_scatter(b)
        convert(b)
        fire_scatter(g, b)
        if refill:
            fire_gather(g + 2, b)

    fire_gather(0, 0)
    fire_gather(1, 1)
    slot(0, 0, False, True)
    slot(1, 1, False, True)

    def body(r, carry):
        for j in range(2):
            slot(2 + r * 2 + j, j, True, True)
        return carry

    lax.fori_loop(0, (_NCHUNK - 4) // 2, body, 0)

    slot(_NCHUNK - 2, 0, True, False)
    slot(_NCHUNK - 1, 1, True, False)
    wait_scatter(0)
    wait_scatter(1)


def kernel(hour, day, weekday, month, W_hour, W_day, W_weekday, W_month):
    table = jnp.concatenate([W_hour, W_day, W_weekday, W_month], axis=0)
    tb = table.astype(jnp.bfloat16)
    packed = jax.lax.bitcast_convert_type(
        jnp.stack([tb[:, :_DW], tb[:, _DW:]], axis=-1), jnp.int32
    )
    packed_rep = jnp.tile(packed, (_NW, 1))
    idx = jnp.stack(
        [
            hour.astype(jnp.int32),
            day.astype(jnp.int32) + 24,
            weekday.astype(jnp.int32) + 55,
            month.astype(jnp.int32) + 62,
        ],
        axis=1,
    ).reshape(_ROWS)
    idx = idx + (jnp.arange(_ROWS, dtype=jnp.int32) // _BPW) * _VROWS
    out = _gather_all(packed_rep, idx)
    return out.reshape(_B, 4 * _D)


# speedup vs baseline: 1.0505x; 1.0373x over previous
import functools

import jax
import jax.numpy as jnp
from jax import lax
from jax.experimental import pallas as pl
from jax.experimental.pallas import tpu as pltpu
from jax.experimental.pallas import tpu_sc as plsc

_B = 16384
_D = 512
_DW = _D // 2
_NW = 32
_ROWS = 4 * _B
_BPW = _ROWS // _NW
_CHUNK = 64
_NCHUNK = _BPW // _CHUNK
_VROWS = 74

_mesh = plsc.VectorSubcoreMesh(core_axis_name="c", subcore_axis_name="s")


@functools.partial(
    pl.kernel,
    mesh=_mesh,
    out_type=jax.ShapeDtypeStruct((_ROWS, _D), jnp.float32),
    scratch_types=[
        pltpu.VMEM((_BPW,), jnp.int32),
        pltpu.VMEM((_CHUNK, _DW), jnp.int32),
        pltpu.VMEM((_CHUNK, _DW), jnp.int32),
        pltpu.VMEM((_CHUNK, _D), jnp.float32),
        pltpu.VMEM((_CHUNK, _D), jnp.float32),
        pltpu.SemaphoreType.DMA,
        pltpu.SemaphoreType.DMA,
    ],
    compiler_params=pltpu.CompilerParams(needs_layout_passes=False),
)
def _gather_all(table_hbm, idx_hbm, out_hbm, idx_v, p0, p1, f0, f1, gsem, ssem):
    pbufs = (p0, p1)
    fbufs = (f0, f1)
    wid = lax.axis_index("s") * 2 + lax.axis_index("c")
    base = wid * _BPW

    pltpu.sync_copy(idx_hbm.at[pl.ds(base, _BPW)], idx_v)

    def fire_gather(g, b):
        pltpu.async_copy(
            table_hbm.at[idx_v.at[pl.ds(g * _CHUNK, _CHUNK)]], pbufs[b], gsem
        )

    def wait_gather(b):
        pltpu.make_async_copy(table_hbm.at[pl.ds(0, _CHUNK)], pbufs[b], gsem).wait()

    _H = _CHUNK // 2

    def fire_scatter(g, b, h):
        pltpu.async_copy(
            fbufs[b].at[pl.ds(h * _H, _H)],
            out_hbm.at[pl.ds(base + g * _CHUNK + h * _H, _H)],
            ssem,
        )

    def wait_scatter(b):
        pltpu.make_async_copy(
            fbufs[b].at[pl.ds(0, _H)], out_hbm.at[pl.ds(base, _H)], ssem
        ).wait()

    def convert(b, h):
        hi_mask = jnp.int32(-65536)

        def col_body(k, carry):
            col = k * 16
            for r in range(h * _H, (h + 1) * _H):
                w = pbufs[b][r, pl.ds(col, 16)]
                lo = plsc.bitcast(w << 16, jnp.float32)
                hi = plsc.bitcast(w & hi_mask, jnp.float32)
                fbufs[b][r, pl.ds(col, 16)] = lo
                fbufs[b][r, pl.ds(_DW + col, 16)] = hi
            return carry

        lax.fori_loop(0, _DW // 16, col_body, 0)

    def slot(g, b, drain, refill):
        wait_gather(b)
        if drain:
            wait_scatter(b)
            wait_scatter(b)
        convert(b, 0)
        fire_scatter(g, b, 0)
        convert(b, 1)
        fire_scatter(g, b, 1)
        if refill:
            fire_gather(g + 2, b)

    fire_gather(0, 0)
    fire_gather(1, 1)
    slot(0, 0, False, True)
    slot(1, 1, False, True)

    def body(r, carry):
        for j in range(2):
            slot(2 + r * 2 + j, j, True, True)
        return carry

    lax.fori_loop(0, (_NCHUNK - 4) // 2, body, 0)

    slot(_NCHUNK - 2, 0, True, False)
    slot(_NCHUNK - 1, 1, True, False)
    wait_scatter(0)
    wait_scatter(0)
    wait_scatter(1)
    wait_scatter(1)


def kernel(hour, day, weekday, month, W_hour, W_day, W_weekday, W_month):
    table = jnp.concatenate([W_hour, W_day, W_weekday, W_month], axis=0)
    tb = table.astype(jnp.bfloat16)
    packed = jax.lax.bitcast_convert_type(
        jnp.stack([tb[:, :_DW], tb[:, _DW:]], axis=-1), jnp.int32
    )
    packed_rep = jnp.tile(packed, (_NW, 1))
    idx = jnp.stack(
        [
            hour.astype(jnp.int32),
            day.astype(jnp.int32) + 24,
            weekday.astype(jnp.int32) + 55,
            month.astype(jnp.int32) + 62,
        ],
        axis=1,
    ).reshape(_ROWS)
    idx = idx + (jnp.arange(_ROWS, dtype=jnp.int32) // _BPW) * _VROWS
    out = _gather_all(packed_rep, idx)
    return out.reshape(_B, 4 * _D)


# speedup vs baseline: 1.0555x; 1.0048x over previous
import functools

import jax
import jax.numpy as jnp
from jax import lax
from jax.experimental import pallas as pl
from jax.experimental.pallas import tpu as pltpu
from jax.experimental.pallas import tpu_sc as plsc

_B = 16384
_D = 512
_DW = _D // 2
_NW = 32
_ROWS = 4 * _B
_BPW = _ROWS // _NW
_CHUNK = 64
_NCHUNK = _BPW // _CHUNK
_VROWS = 74

_mesh = plsc.VectorSubcoreMesh(core_axis_name="c", subcore_axis_name="s")


@functools.partial(
    pl.kernel,
    mesh=_mesh,
    out_type=jax.ShapeDtypeStruct((_ROWS, _D), jnp.float32),
    scratch_types=[
        pltpu.VMEM((_BPW,), jnp.int32),
        pltpu.VMEM((_CHUNK, _DW), jnp.int32),
        pltpu.VMEM((_CHUNK, _DW), jnp.int32),
        pltpu.VMEM((_CHUNK, _D), jnp.float32),
        pltpu.VMEM((_CHUNK, _D), jnp.float32),
        pltpu.SemaphoreType.DMA,
        pltpu.SemaphoreType.DMA,
    ],
    compiler_params=pltpu.CompilerParams(needs_layout_passes=False),
)
def _gather_all(table_hbm, idx_hbm, out_hbm, idx_v, p0, p1, f0, f1, gsem, ssem):
    pbufs = (p0, p1)
    fbufs = (f0, f1)
    wid = lax.axis_index("s") * 2 + lax.axis_index("c")
    base = wid * _BPW

    pltpu.sync_copy(idx_hbm.at[pl.ds(base, _BPW)], idx_v)

    def fire_gather(g, b):
        pltpu.async_copy(
            table_hbm.at[idx_v.at[pl.ds(g * _CHUNK, _CHUNK)]], pbufs[b], gsem
        )

    def wait_gather(b):
        pltpu.make_async_copy(table_hbm.at[pl.ds(0, _CHUNK)], pbufs[b], gsem).wait()

    _H = _CHUNK // 4

    def fire_scatter(g, b, h):
        pltpu.async_copy(
            fbufs[b].at[pl.ds(h * _H, _H)],
            out_hbm.at[pl.ds(base + g * _CHUNK + h * _H, _H)],
            ssem,
        )

    def wait_scatter(b):
        pltpu.make_async_copy(
            fbufs[b].at[pl.ds(0, _H)], out_hbm.at[pl.ds(base, _H)], ssem
        ).wait()

    def convert(b, h):
        hi_mask = jnp.int32(-65536)

        def col_body(k, carry):
            col = k * 16
            for r in range(h * _H, (h + 1) * _H):
                w = pbufs[b][r, pl.ds(col, 16)]
                lo = plsc.bitcast(w << 16, jnp.float32)
                hi = plsc.bitcast(w & hi_mask, jnp.float32)
                fbufs[b][r, pl.ds(col, 16)] = lo
                fbufs[b][r, pl.ds(_DW + col, 16)] = hi
            return carry

        lax.fori_loop(0, _DW // 16, col_body, 0)

    def slot(g, b, drain, refill):
        wait_gather(b)
        if drain:
            for _ in range(4):
                wait_scatter(b)
        for h in range(4):
            convert(b, h)
            fire_scatter(g, b, h)
        if refill:
            fire_gather(g + 2, b)

    fire_gather(0, 0)
    fire_gather(1, 1)
    slot(0, 0, False, True)
    slot(1, 1, False, True)

    def body(r, carry):
        for j in range(2):
            slot(2 + r * 2 + j, j, True, True)
        return carry

    lax.fori_loop(0, (_NCHUNK - 4) // 2, body, 0)

    slot(_NCHUNK - 2, 0, True, False)
    slot(_NCHUNK - 1, 1, True, False)
    for _ in range(4):
        wait_scatter(0)
    for _ in range(4):
        wait_scatter(1)


def kernel(hour, day, weekday, month, W_hour, W_day, W_weekday, W_month):
    table = jnp.concatenate([W_hour, W_day, W_weekday, W_month], axis=0)
    tb = table.astype(jnp.bfloat16)
    packed = jax.lax.bitcast_convert_type(
        jnp.stack([tb[:, :_DW], tb[:, _DW:]], axis=-1), jnp.int32
    )
    packed_rep = jnp.tile(packed, (_NW, 1))
    idx = jnp.stack(
        [
            hour.astype(jnp.int32),
            day.astype(jnp.int32) + 24,
            weekday.astype(jnp.int32) + 55,
            month.astype(jnp.int32) + 62,
        ],
        axis=1,
    ).reshape(_ROWS)
    idx = idx + (jnp.arange(_ROWS, dtype=jnp.int32) // _BPW) * _VROWS
    out = _gather_all(packed_rep, idx)
    return out.reshape(_B, 4 * _D)
